# speedup vs baseline: 1.9073x; 1.9073x over previous
import functools

import jax
import jax.numpy as jnp
from jax import lax
from jax.experimental import pallas as pl
from jax.experimental.pallas import tpu as pltpu
from jax.experimental.pallas import tpu_sc as plsc

_B, _N, _D = 8, 50000, 128
_K = 200
_KP = 256
_NC, _NS = 2, 16
_NW = _NC * _NS
_BPW = (_B * _KP) // _NW
_WPB = _KP // _BPW

_mesh = plsc.VectorSubcoreMesh(core_axis_name="c", subcore_axis_name="s")


@functools.partial(
    pl.kernel,
    mesh=_mesh,
    out_type=jax.ShapeDtypeStruct((_B * _KP, _D), jnp.float32),
    scratch_types=[
        pltpu.VMEM((_BPW,), jnp.int32),
        pltpu.VMEM((_BPW, _D), jnp.float32),
        pltpu.SemaphoreType.DMA,
    ],
)
def _sc_gather(table_hbm, idx_hbm, out_hbm, idx_v, rows_v, sem):
    wid = lax.axis_index("s") * _NC + lax.axis_index("c")
    base = wid * _BPW
    pltpu.sync_copy(idx_hbm.at[pl.ds(base, _BPW)], idx_v)
    row_off = (wid // _WPB) * _N
    for j in range(_BPW // 16):
        sl = pl.ds(j * 16, 16)
        idx_v[sl] = idx_v[sl] + row_off
    pltpu.async_copy(table_hbm.at[idx_v], rows_v, sem).wait()
    pltpu.sync_copy(rows_v, out_hbm.at[pl.ds(base, _BPW)])


def kernel(feature, tail_id):
    table = feature.reshape(_B * _N, _D)
    idx = jnp.zeros((_B, _KP), jnp.int32).at[:, :_K].set(tail_id)
    out = _sc_gather(table, idx.reshape(_B * _KP))
    res = out.reshape(_B, _KP, _D)[:, :_K, :]
    return (res, res)


# speedup vs baseline: 2.3546x; 1.2345x over previous
import functools

import jax
import jax.numpy as jnp
from jax import lax
from jax.experimental import pallas as pl
from jax.experimental.pallas import tpu as pltpu
from jax.experimental.pallas import tpu_sc as plsc

_B, _N, _D = 8, 50000, 128
_K = 200
_NC, _NS = 2, 16
_WPB = 4
_CHUNK = 64
_TAIL = _K - 3 * _CHUNK

_mesh = plsc.VectorSubcoreMesh(core_axis_name="c", subcore_axis_name="s")


@functools.partial(
    pl.kernel,
    mesh=_mesh,
    out_type=(
        jax.ShapeDtypeStruct((_B * _K, _D), jnp.float32),
        jax.ShapeDtypeStruct((_B * _K, _D), jnp.float32),
    ),
    scratch_types=[
        pltpu.VMEM((_CHUNK,), jnp.int32),
        pltpu.VMEM((_CHUNK, _D), jnp.float32),
        pltpu.SemaphoreType.DMA,
    ],
)
def _sc_gather(table_hbm, idx_hbm, out_a, out_b, idx_v, rows_v, sem):
    wid = lax.axis_index("s") * _NC + lax.axis_index("c")
    b = wid // _WPB
    w4 = wid % _WPB
    row_off = b * _N

    @pl.when(w4 < _WPB - 1)
    def _full_chunk():
        base = b * _K + w4 * _CHUNK
        pltpu.sync_copy(idx_hbm.at[pl.ds(base, _CHUNK)], idx_v)
        for j in range(_CHUNK // 16):
            sl = pl.ds(j * 16, 16)
            idx_v[sl] = idx_v[sl] + row_off
        pltpu.async_copy(table_hbm.at[idx_v], rows_v, sem).wait()
        pltpu.sync_copy(rows_v, out_a.at[pl.ds(base, _CHUNK)])
        pltpu.sync_copy(rows_v, out_b.at[pl.ds(base, _CHUNK)])

    @pl.when(w4 == _WPB - 1)
    def _tail_chunk():
        base = b * _K + _K - 16
        pltpu.sync_copy(idx_hbm.at[pl.ds(base, 16)], idx_v.at[pl.ds(0, 16)])
        idx_v[pl.ds(0, 16)] = idx_v[pl.ds(0, 16)] + row_off
        pltpu.async_copy(
            table_hbm.at[idx_v.at[pl.ds(0, 16)]], rows_v.at[pl.ds(0, 16)], sem
        ).wait()
        out_base = b * _K + _K - _TAIL
        src = rows_v.at[pl.ds(16 - _TAIL, _TAIL)]
        pltpu.sync_copy(src, out_a.at[pl.ds(out_base, _TAIL)])
        pltpu.sync_copy(src, out_b.at[pl.ds(out_base, _TAIL)])


def kernel(feature, tail_id):
    table = feature.reshape(_B * _N, _D)
    out_a, out_b = _sc_gather(table, tail_id.reshape(_B * _K))
    shape = (_B, _K, _D)
    return (out_a.reshape(shape), out_b.reshape(shape))
